# baseline (speedup 1.0000x reference)
import functools

import jax
import jax.numpy as jnp
import numpy as np
from jax import lax
from jax.experimental import pallas as pl
from jax.experimental.pallas import tpu as pltpu

RHO_ = 5.0
BS_ = 256
NHASH_ = 64
NLAB_ = 14
NVALS_ = NLAB_ + 1
QB_ = 64
NACC_ = 8

def _loss_body(h_ref, l_ref, out_ref, dgt_ref, sb_ref, spb_ref, rz_ref):
    lab = l_ref[:]
    dgt = lax.dot_general(lab, lab, (((1,), (1,)), ((), ())),
                          preferred_element_type=jnp.float32)
    rows = lax.broadcasted_iota(jnp.int32, (BS_, BS_), 0)
    cols = lax.broadcasted_iota(jnp.int32, (BS_, BS_), 1)
    dgt = jnp.where(rows == cols, 0.0, dgt)
    h = h_ref[:]
    gram = lax.dot_general(h, h, (((1,), (1,)), ((), ())),
                           preferred_element_type=jnp.float32)
    dsim = 0.5 * (jnp.float32(NHASH_) - gram)
    dgt_ref[:] = dgt.astype(jnp.bfloat16)
    sb_ref[:] = dsim.astype(jnp.bfloat16)
    spb_ref[:] = (dsim + RHO_).astype(jnp.bfloat16)

    cum_le = jnp.zeros((1, BS_), jnp.float32)
    z = jnp.zeros((1, BS_), jnp.float32)
    cnt = jnp.zeros((1, BS_), jnp.float32)
    prev1 = jnp.zeros((1, BS_), jnp.float32)
    prev2 = jnp.zeros((1, BS_), jnp.float32)
    for v in range(NVALS_):
        cv = jnp.sum(jnp.where(dgt == jnp.float32(v), 1.0, 0.0),
                     axis=0, keepdims=True)
        c_lt = cum_le
        cum_le = cum_le + cv
        c_gt = jnp.float32(BS_) - cum_le
        z = z + (2.0 ** v) * cv * (c_lt - c_gt)
        cnt = cnt + cv * (prev1 + prev2)
        prev2 = prev1
        prev1 = cv

    rz_row = jnp.where(z > 0.0, 1.0 / jnp.where(z > 0.0, z, 1.0), 0.0)
    rz_ref[:] = jnp.transpose(rz_row)

    vals16b = lax.broadcasted_iota(jnp.int32, (1, 16, 1), 1).astype(jnp.bfloat16)
    a_i = lax.broadcasted_iota(jnp.int32, (16, 16), 0).astype(jnp.float32)
    b_i = lax.broadcasted_iota(jnp.int32, (16, 16), 1).astype(jnp.float32)
    d_ab = a_i - b_i
    wmat = jnp.where((d_ab >= 0.5) & (d_ab <= 2.5),
                     jnp.exp2(a_i) - jnp.exp2(b_i),
                     0.0).astype(jnp.bfloat16)

    def per_block(blk, accs):
        q0 = blk * QB_
        d_blk = dgt_ref[pl.ds(q0, QB_), :]
        rz_blk = rz_ref[pl.ds(q0, QB_), :]
        sp_blk = spb_ref[pl.ds(q0, QB_), :]
        s_blk = sb_ref[pl.ds(q0, QB_), :]
        r3 = jnp.maximum(sp_blk[:, :, None] - s_blk[:, None, :],
                         jnp.bfloat16(0.0))
        ut3 = (d_blk[:, None, :] == vals16b).astype(jnp.bfloat16)
        accs = list(accs)
        for qq in range(QB_):
            ut_b = ut3[qq]
            p = lax.dot_general(ut_b, r3[qq], (((1,), (0,)), ((), ())),
                                preferred_element_type=jnp.float32)
            v = lax.dot_general(wmat, ut_b, (((1,), (0,)), ((), ())),
                                preferred_element_type=jnp.float32)
            k = qq % NACC_
            accs[k] = accs[k] + (p * v) * rz_blk[qq:qq + 1, :]
        return tuple(accs)

    accs = lax.fori_loop(0, BS_ // QB_, per_block,
                         tuple(jnp.zeros((16, BS_), jnp.float32)
                               for _ in range(NACC_)))

    losses = jnp.sum(sum(accs[1:], accs[0]))
    counts = jnp.sum(cnt)
    total = jnp.where(counts > 0.0, losses / counts, losses)
    out_ref[:, :] = jnp.broadcast_to(total, (1, 1))


@jax.jit
def kernel(hash_features, labels):
    out = pl.pallas_call(
        _loss_body,
        out_shape=jax.ShapeDtypeStruct((1, 1), jnp.float32),
        scratch_shapes=[
            pltpu.VMEM((BS_, BS_), jnp.bfloat16),
            pltpu.VMEM((BS_, BS_), jnp.bfloat16),
            pltpu.VMEM((BS_, BS_), jnp.bfloat16),
            pltpu.VMEM((BS_, 1), jnp.float32),
        ],
    )(hash_features, labels)
    return jnp.reshape(out, (1,))


# speedup vs baseline: 1.0376x; 1.0376x over previous
import functools

import jax
import jax.numpy as jnp
import numpy as np
from jax import lax
from jax.experimental import pallas as pl
from jax.experimental.pallas import tpu as pltpu

RHO_ = 5.0
BS_ = 256
NHASH_ = 64
NLAB_ = 14
NVALS_ = NLAB_ + 1
QB_ = 64
NACC_ = 8

def _loss_body(h_ref, l_ref, out_ref, dgt_ref, sb_ref, spb_ref, rz_ref):
    lab = l_ref[:]
    dgt = lax.dot_general(lab, lab, (((1,), (1,)), ((), ())),
                          preferred_element_type=jnp.float32)
    rows = lax.broadcasted_iota(jnp.int32, (BS_, BS_), 0)
    cols = lax.broadcasted_iota(jnp.int32, (BS_, BS_), 1)
    dgt = jnp.where(rows == cols, 0.0, dgt)
    h = h_ref[:]
    gram = lax.dot_general(h, h, (((1,), (1,)), ((), ())),
                           preferred_element_type=jnp.float32)
    dsim = 0.5 * (jnp.float32(NHASH_) - gram)
    dgt_ref[:] = dgt.astype(jnp.bfloat16)
    sb_ref[:] = dsim.astype(jnp.bfloat16)
    spb_ref[:] = (dsim + RHO_).astype(jnp.bfloat16)

    cum_le = jnp.zeros((1, BS_), jnp.float32)
    z = jnp.zeros((1, BS_), jnp.float32)
    cnt = jnp.zeros((1, BS_), jnp.float32)
    prev1 = jnp.zeros((1, BS_), jnp.float32)
    prev2 = jnp.zeros((1, BS_), jnp.float32)
    for v in range(NVALS_):
        cv = jnp.sum(jnp.where(dgt == jnp.float32(v), 1.0, 0.0),
                     axis=0, keepdims=True)
        c_lt = cum_le
        cum_le = cum_le + cv
        c_gt = jnp.float32(BS_) - cum_le
        z = z + (2.0 ** v) * cv * (c_lt - c_gt)
        cnt = cnt + cv * (prev1 + prev2)
        prev2 = prev1
        prev1 = cv

    rz_row = jnp.where(z > 0.0, 1.0 / jnp.where(z > 0.0, z, 1.0), 0.0)
    rz_ref[:] = jnp.transpose(rz_row)

    vals16b = lax.broadcasted_iota(jnp.int32, (1, 16, 1), 1).astype(jnp.bfloat16)
    vals16 = lax.broadcasted_iota(jnp.int32, (16, 1), 0).astype(jnp.float32)
    vals16e = jnp.exp2(vals16)

    def per_block(blk, accs):
        q0 = blk * QB_
        d_blk = dgt_ref[pl.ds(q0, QB_), :]
        d_blk_f = d_blk.astype(jnp.float32)
        e_blk = jnp.exp2(d_blk_f)
        rz_blk = rz_ref[pl.ds(q0, QB_), :]
        sp_blk = spb_ref[pl.ds(q0, QB_), :]
        s_blk = sb_ref[pl.ds(q0, QB_), :]
        r3 = jnp.maximum(sp_blk[:, :, None] - s_blk[:, None, :],
                         jnp.bfloat16(0.0))
        ut3 = (d_blk[:, None, :] == vals16b).astype(jnp.bfloat16)
        accs = list(accs)
        for qq in range(QB_):
            ut_b = ut3[qq]
            p = lax.dot_general(ut_b, r3[qq], (((1,), (0,)), ((), ())),
                                preferred_element_type=jnp.float32)
            adiff = vals16 - d_blk_f[qq:qq + 1, :]
            v = jnp.where((adiff >= 0.5) & (adiff <= 2.5),
                          vals16e - e_blk[qq:qq + 1, :], 0.0)
            k = qq % NACC_
            accs[k] = accs[k] + (p * v) * rz_blk[qq:qq + 1, :]
        return tuple(accs)

    accs = lax.fori_loop(0, BS_ // QB_, per_block,
                         tuple(jnp.zeros((16, BS_), jnp.float32)
                               for _ in range(NACC_)))

    losses = jnp.sum(sum(accs[1:], accs[0]))
    counts = jnp.sum(cnt)
    total = jnp.where(counts > 0.0, losses / counts, losses)
    out_ref[:, :] = jnp.broadcast_to(total, (1, 1))


@jax.jit
def kernel(hash_features, labels):
    out = pl.pallas_call(
        _loss_body,
        out_shape=jax.ShapeDtypeStruct((1, 1), jnp.float32),
        scratch_shapes=[
            pltpu.VMEM((BS_, BS_), jnp.bfloat16),
            pltpu.VMEM((BS_, BS_), jnp.bfloat16),
            pltpu.VMEM((BS_, BS_), jnp.bfloat16),
            pltpu.VMEM((BS_, 1), jnp.float32),
        ],
    )(hash_features, labels)
    return jnp.reshape(out, (1,))


# speedup vs baseline: 1.0633x; 1.0248x over previous
import functools

import jax
import jax.numpy as jnp
import numpy as np
from jax import lax
from jax.experimental import pallas as pl
from jax.experimental.pallas import tpu as pltpu

RHO_ = 5.0
BS_ = 256
NHASH_ = 64
NLAB_ = 14
NVALS_ = NLAB_ + 1
QB_ = 128
NACC_ = 8

def _loss_body(h_ref, l_ref, out_ref, dgt_ref, sb_ref, spb_ref, rz_ref):
    lab = l_ref[:]
    dgt = lax.dot_general(lab, lab, (((1,), (1,)), ((), ())),
                          preferred_element_type=jnp.float32)
    rows = lax.broadcasted_iota(jnp.int32, (BS_, BS_), 0)
    cols = lax.broadcasted_iota(jnp.int32, (BS_, BS_), 1)
    dgt = jnp.where(rows == cols, 0.0, dgt)
    h = h_ref[:]
    gram = lax.dot_general(h, h, (((1,), (1,)), ((), ())),
                           preferred_element_type=jnp.float32)
    dsim = 0.5 * (jnp.float32(NHASH_) - gram)
    dgt_ref[:] = dgt.astype(jnp.bfloat16)
    sb_ref[:] = dsim.astype(jnp.bfloat16)
    spb_ref[:] = (dsim + RHO_).astype(jnp.bfloat16)

    cum_le = jnp.zeros((1, BS_), jnp.float32)
    z = jnp.zeros((1, BS_), jnp.float32)
    cnt = jnp.zeros((1, BS_), jnp.float32)
    prev1 = jnp.zeros((1, BS_), jnp.float32)
    prev2 = jnp.zeros((1, BS_), jnp.float32)
    for v in range(NVALS_):
        cv = jnp.sum(jnp.where(dgt == jnp.float32(v), 1.0, 0.0),
                     axis=0, keepdims=True)
        c_lt = cum_le
        cum_le = cum_le + cv
        c_gt = jnp.float32(BS_) - cum_le
        z = z + (2.0 ** v) * cv * (c_lt - c_gt)
        cnt = cnt + cv * (prev1 + prev2)
        prev2 = prev1
        prev1 = cv

    rz_row = jnp.where(z > 0.0, 1.0 / jnp.where(z > 0.0, z, 1.0), 0.0)
    rz_ref[:] = jnp.transpose(rz_row)

    vals16b = lax.broadcasted_iota(jnp.int32, (1, 16, 1), 1).astype(jnp.bfloat16)
    vals16 = lax.broadcasted_iota(jnp.int32, (16, 1), 0).astype(jnp.float32)
    vals16e = jnp.exp2(vals16)

    def per_block(blk, accs):
        q0 = blk * QB_
        d_blk = dgt_ref[pl.ds(q0, QB_), :]
        d_blk_f = d_blk.astype(jnp.float32)
        e_blk = jnp.exp2(d_blk_f)
        rz_blk = rz_ref[pl.ds(q0, QB_), :]
        sp_blk = spb_ref[pl.ds(q0, QB_), :]
        s_blk = sb_ref[pl.ds(q0, QB_), :]
        r3 = jnp.maximum(sp_blk[:, :, None] - s_blk[:, None, :],
                         jnp.bfloat16(0.0))
        ut3 = (d_blk[:, None, :] == vals16b).astype(jnp.bfloat16)
        accs = list(accs)
        for qq in range(QB_):
            ut_b = ut3[qq]
            p = lax.dot_general(ut_b, r3[qq], (((1,), (0,)), ((), ())),
                                preferred_element_type=jnp.float32)
            adiff = vals16 - d_blk_f[qq:qq + 1, :]
            v = jnp.where((adiff >= 0.5) & (adiff <= 2.5),
                          vals16e - e_blk[qq:qq + 1, :], 0.0)
            k = qq % NACC_
            accs[k] = accs[k] + (p * v) * rz_blk[qq:qq + 1, :]
        return tuple(accs)

    accs = lax.fori_loop(0, BS_ // QB_, per_block,
                         tuple(jnp.zeros((16, BS_), jnp.float32)
                               for _ in range(NACC_)))

    losses = jnp.sum(sum(accs[1:], accs[0]))
    counts = jnp.sum(cnt)
    total = jnp.where(counts > 0.0, losses / counts, losses)
    out_ref[:, :] = jnp.broadcast_to(total, (1, 1))


@jax.jit
def kernel(hash_features, labels):
    out = pl.pallas_call(
        _loss_body,
        out_shape=jax.ShapeDtypeStruct((1, 1), jnp.float32),
        scratch_shapes=[
            pltpu.VMEM((BS_, BS_), jnp.bfloat16),
            pltpu.VMEM((BS_, BS_), jnp.bfloat16),
            pltpu.VMEM((BS_, BS_), jnp.bfloat16),
            pltpu.VMEM((BS_, 1), jnp.float32),
        ],
    )(hash_features, labels)
    return jnp.reshape(out, (1,))


# speedup vs baseline: 1.1207x; 1.0540x over previous
import functools

import jax
import jax.numpy as jnp
import numpy as np
from jax import lax
from jax.experimental import pallas as pl
from jax.experimental.pallas import tpu as pltpu

RHO_ = 5.0
BS_ = 256
NHASH_ = 64
NLAB_ = 14
NVALS_ = NLAB_ + 1
QB_ = 128


def _loss_body(h_ref, l_ref, out_ref, dgt_ref, sb_ref, spb_ref, num_ref):
    lab = l_ref[:]
    dgt = lax.dot_general(lab, lab, (((1,), (1,)), ((), ())),
                          preferred_element_type=jnp.float32)
    rows = lax.broadcasted_iota(jnp.int32, (BS_, BS_), 0)
    cols = lax.broadcasted_iota(jnp.int32, (BS_, BS_), 1)
    dgt = jnp.where(rows == cols, 0.0, dgt)
    h = h_ref[:]
    gram = lax.dot_general(h, h, (((1,), (1,)), ((), ())),
                           preferred_element_type=jnp.float32)
    dsim = 0.5 * (jnp.float32(NHASH_) - gram)
    dgt_ref[:] = dgt
    sb_ref[:] = dsim.astype(jnp.bfloat16)
    spb_ref[:] = (dsim + RHO_).astype(jnp.bfloat16)

    cum_le = jnp.zeros((BS_, 1), jnp.float32)
    z = jnp.zeros((BS_, 1), jnp.float32)
    cnt = jnp.zeros((BS_, 1), jnp.float32)
    prev1 = jnp.zeros((BS_, 1), jnp.float32)
    prev2 = jnp.zeros((BS_, 1), jnp.float32)
    for v in range(NVALS_):
        cv = jnp.sum(jnp.where(dgt == jnp.float32(v), 1.0, 0.0),
                     axis=1, keepdims=True)
        c_lt = cum_le
        cum_le = cum_le + cv
        c_gt = jnp.float32(BS_) - cum_le
        z = z + (2.0 ** v) * cv * (c_lt - c_gt)
        cnt = cnt + cv * (prev1 + prev2)
        prev2 = prev1
        prev1 = cv

    vals16 = lax.broadcasted_iota(jnp.int32, (16, 1), 0).astype(jnp.float32)
    vals16e = jnp.exp2(vals16)

    def per_block(blk, carry):
        q0 = blk * QB_
        d_blk = dgt_ref[pl.ds(q0, QB_), :]
        e_blk = jnp.exp2(d_blk)
        sp_blk = spb_ref[pl.ds(q0, QB_), :]
        s_blk = sb_ref[pl.ds(q0, QB_), :]
        r3 = jnp.maximum(sp_blk[:, :, None] - s_blk[:, None, :],
                         jnp.bfloat16(0.0))
        nums = []
        for qq in range(QB_):
            d_row = d_blk[qq:qq + 1, :]
            ut_b = (d_row == vals16).astype(jnp.bfloat16)
            p = lax.dot_general(ut_b, r3[qq], (((1,), (0,)), ((), ())),
                                preferred_element_type=jnp.float32)
            adiff = vals16 - d_row
            v = jnp.where((adiff >= 0.5) & (adiff <= 2.5),
                          vals16e - e_blk[qq:qq + 1, :], 0.0)
            nums.append(jnp.broadcast_to(jnp.sum(p * v), (1, 1)))
        num_ref[pl.ds(q0, QB_), :] = jnp.concatenate(nums, axis=0)
        return carry

    lax.fori_loop(0, BS_ // QB_, per_block, 0)

    num = num_ref[:]
    per_idx = jnp.where(z > 0.0, num / jnp.where(z > 0.0, z, 1.0), 0.0)
    losses = jnp.sum(per_idx)
    counts = jnp.sum(cnt)
    total = jnp.where(counts > 0.0, losses / counts, losses)
    out_ref[:, :] = jnp.broadcast_to(total, (1, 1))


@jax.jit
def kernel(hash_features, labels):
    out = pl.pallas_call(
        _loss_body,
        out_shape=jax.ShapeDtypeStruct((1, 1), jnp.float32),
        scratch_shapes=[
            pltpu.VMEM((BS_, BS_), jnp.float32),
            pltpu.VMEM((BS_, BS_), jnp.bfloat16),
            pltpu.VMEM((BS_, BS_), jnp.bfloat16),
            pltpu.VMEM((BS_, 1), jnp.float32),
        ],
    )(hash_features, labels)
    return jnp.reshape(out, (1,))


# speedup vs baseline: 1.1236x; 1.0025x over previous
import jax
import jax.numpy as jnp
from jax import lax
from jax.experimental import pallas as pl
from jax.experimental.pallas import tpu as pltpu

RHO_ = 5.0
BS_ = 256
NHASH_ = 64
NLAB_ = 14
NVALS_ = NLAB_ + 1
QB_ = 128


def _loss_body(h_ref, l_ref, out_ref, dgt_ref, sb_ref, spb_ref, num_ref):
    lab = l_ref[:]
    dgt = lax.dot_general(lab, lab, (((1,), (1,)), ((), ())),
                          preferred_element_type=jnp.float32)
    rows = lax.broadcasted_iota(jnp.int32, (BS_, BS_), 0)
    cols = lax.broadcasted_iota(jnp.int32, (BS_, BS_), 1)
    dgt = jnp.where(rows == cols, 0.0, dgt)
    h = h_ref[:]
    gram = lax.dot_general(h, h, (((1,), (1,)), ((), ())),
                           preferred_element_type=jnp.float32)
    dsim = 0.5 * (jnp.float32(NHASH_) - gram)
    dgt_ref[:] = dgt
    sb_ref[:] = dsim.astype(jnp.bfloat16)
    spb_ref[:] = (dsim + RHO_).astype(jnp.bfloat16)

    cum_le = jnp.zeros((BS_, 1), jnp.float32)
    z = jnp.zeros((BS_, 1), jnp.float32)
    cnt = jnp.zeros((BS_, 1), jnp.float32)
    prev1 = jnp.zeros((BS_, 1), jnp.float32)
    prev2 = jnp.zeros((BS_, 1), jnp.float32)
    for v in range(NVALS_):
        cv = jnp.sum(jnp.where(dgt == jnp.float32(v), 1.0, 0.0),
                     axis=1, keepdims=True)
        c_lt = cum_le
        cum_le = cum_le + cv
        c_gt = jnp.float32(BS_) - cum_le
        z = z + (2.0 ** v) * cv * (c_lt - c_gt)
        cnt = cnt + cv * (prev1 + prev2)
        prev2 = prev1
        prev1 = cv

    vals16 = lax.broadcasted_iota(jnp.int32, (16, 1), 0).astype(jnp.float32)
    vals16b = vals16.astype(jnp.bfloat16)
    vals16eb = jnp.exp2(vals16).astype(jnp.bfloat16)

    def per_block(blk, carry):
        q0 = blk * QB_
        d_blk = dgt_ref[pl.ds(q0, QB_), :]
        d_blk_b = d_blk.astype(jnp.bfloat16)
        e_blk_b = jnp.exp2(d_blk).astype(jnp.bfloat16)
        sp_blk = spb_ref[pl.ds(q0, QB_), :]
        s_blk = sb_ref[pl.ds(q0, QB_), :]
        r3 = jnp.maximum(sp_blk[:, :, None] - s_blk[:, None, :],
                         jnp.bfloat16(0.0))
        nums = []
        for qq in range(QB_):
            d_row = d_blk_b[qq:qq + 1, :]
            ut_b = (d_row == vals16b).astype(jnp.bfloat16)
            p = lax.dot_general(ut_b, r3[qq], (((1,), (0,)), ((), ())),
                                preferred_element_type=jnp.float32)
            adiff = vals16b - d_row
            v = jnp.where((adiff >= jnp.bfloat16(0.5)) &
                          (adiff <= jnp.bfloat16(2.5)),
                          vals16eb - e_blk_b[qq:qq + 1, :],
                          jnp.bfloat16(0.0))
            nums.append(jnp.broadcast_to(
                jnp.sum(p * v.astype(jnp.float32)), (1, 1)))
        num_ref[pl.ds(q0, QB_), :] = jnp.concatenate(nums, axis=0)
        return carry

    lax.fori_loop(0, BS_ // QB_, per_block, 0)

    num = num_ref[:]
    per_idx = jnp.where(z > 0.0, num / jnp.where(z > 0.0, z, 1.0), 0.0)
    losses = jnp.sum(per_idx)
    counts = jnp.sum(cnt)
    total = jnp.where(counts > 0.0, losses / counts, losses)
    out_ref[:, :] = jnp.broadcast_to(total, (1, 1))


@jax.jit
def kernel(hash_features, labels):
    out = pl.pallas_call(
        _loss_body,
        out_shape=jax.ShapeDtypeStruct((1, 1), jnp.float32),
        scratch_shapes=[
            pltpu.VMEM((BS_, BS_), jnp.float32),
            pltpu.VMEM((BS_, BS_), jnp.bfloat16),
            pltpu.VMEM((BS_, BS_), jnp.bfloat16),
            pltpu.VMEM((BS_, 1), jnp.float32),
        ],
    )(hash_features, labels)
    return jnp.reshape(out, (1,))


# speedup vs baseline: 1.1628x; 1.0349x over previous
import jax
import jax.numpy as jnp
from jax import lax
from jax.experimental import pallas as pl
from jax.experimental.pallas import tpu as pltpu

RHO_ = 5.0
BS_ = 256
NHASH_ = 64
NLAB_ = 14
NVALS_ = NLAB_ + 1
QB_ = 128


def _loss_body(h_ref, l_ref, out_ref, dgt_ref, sb_ref, spb_ref, num_ref):
    lab = l_ref[:]
    dgt = lax.dot_general(lab, lab, (((1,), (1,)), ((), ())),
                          preferred_element_type=jnp.float32)
    rows = lax.broadcasted_iota(jnp.int32, (BS_, BS_), 0)
    cols = lax.broadcasted_iota(jnp.int32, (BS_, BS_), 1)
    dgt = jnp.where(rows == cols, 0.0, dgt)
    h = h_ref[:]
    gram = lax.dot_general(h, h, (((1,), (1,)), ((), ())),
                           preferred_element_type=jnp.float32)
    dsim = 0.5 * (jnp.float32(NHASH_) - gram)
    dgt_ref[:] = dgt
    sb_ref[:] = dsim.astype(jnp.bfloat16)
    spb_ref[:] = (dsim + RHO_).astype(jnp.bfloat16)

    cum_le = jnp.zeros((1, BS_), jnp.float32)
    z_row = jnp.zeros((1, BS_), jnp.float32)
    cnt = jnp.zeros((1, BS_), jnp.float32)
    prev1 = jnp.zeros((1, BS_), jnp.float32)
    prev2 = jnp.zeros((1, BS_), jnp.float32)
    for v in range(NVALS_):
        cv = jnp.sum(jnp.where(dgt == jnp.float32(v), 1.0, 0.0),
                     axis=0, keepdims=True)
        c_lt = cum_le
        cum_le = cum_le + cv
        c_gt = jnp.float32(BS_) - cum_le
        z_row = z_row + (2.0 ** v) * cv * (c_lt - c_gt)
        cnt = cnt + cv * (prev1 + prev2)
        prev2 = prev1
        prev1 = cv
    z = jnp.transpose(z_row)

    vals16 = lax.broadcasted_iota(jnp.int32, (16, 1), 0).astype(jnp.float32)
    vals16b = vals16.astype(jnp.bfloat16)
    vals16eb = jnp.exp2(vals16).astype(jnp.bfloat16)

    def per_block(blk, carry):
        q0 = blk * QB_
        d_blk = dgt_ref[pl.ds(q0, QB_), :]
        d_blk_b = d_blk.astype(jnp.bfloat16)
        e_blk_b = jnp.exp2(d_blk).astype(jnp.bfloat16)
        sp_blk = spb_ref[pl.ds(q0, QB_), :]
        s_blk = sb_ref[pl.ds(q0, QB_), :]
        r3 = jnp.maximum(sp_blk[:, :, None] - s_blk[:, None, :],
                         jnp.bfloat16(0.0))
        nums = []
        for qq in range(QB_):
            d_row = d_blk_b[qq:qq + 1, :]
            ut_b = (d_row == vals16b).astype(jnp.bfloat16)
            p = lax.dot_general(ut_b, r3[qq], (((1,), (0,)), ((), ())),
                                preferred_element_type=jnp.float32)
            adiff = vals16b - d_row
            v = jnp.where((adiff >= jnp.bfloat16(0.5)) &
                          (adiff <= jnp.bfloat16(2.5)),
                          vals16eb - e_blk_b[qq:qq + 1, :],
                          jnp.bfloat16(0.0))
            nums.append(jnp.broadcast_to(
                jnp.sum(p * v.astype(jnp.float32)), (1, 1)))
        num_ref[pl.ds(q0, QB_), :] = jnp.concatenate(nums, axis=0)
        return carry

    lax.fori_loop(0, BS_ // QB_, per_block, 0)

    num = num_ref[:]
    per_idx = jnp.where(z > 0.0, num / jnp.where(z > 0.0, z, 1.0), 0.0)
    losses = jnp.sum(per_idx)
    counts = jnp.sum(cnt)
    total = jnp.where(counts > 0.0, losses / counts, losses)
    out_ref[:, :] = jnp.broadcast_to(total, (1, 1))


@jax.jit
def kernel(hash_features, labels):
    out = pl.pallas_call(
        _loss_body,
        out_shape=jax.ShapeDtypeStruct((1, 1), jnp.float32),
        scratch_shapes=[
            pltpu.VMEM((BS_, BS_), jnp.float32),
            pltpu.VMEM((BS_, BS_), jnp.bfloat16),
            pltpu.VMEM((BS_, BS_), jnp.bfloat16),
            pltpu.VMEM((BS_, 1), jnp.float32),
        ],
    )(hash_features, labels)
    return jnp.reshape(out, (1,))
